# Initial kernel scaffold; baseline (speedup 1.0000x reference)
import jax
import jax.numpy as jnp
from jax.experimental import pallas as pl


def kernel(x, edge_index, edge_weight, W1, b1, W2, b2):
    raise NotImplementedError("write your pallas kernel here")


# speedup vs baseline: 2.6052x; 2.6052x over previous
import functools
import jax
import jax.numpy as jnp
from jax.experimental import pallas as pl


def _mm_body(x_ref, w_ref, o_ref):
    o_ref[...] = jnp.dot(x_ref[...], w_ref[...],
                         preferred_element_type=jnp.float32)


def _matmul(x, w, bm=1000):
    m, k = x.shape
    _, n = w.shape
    grid = (m // bm,)
    return pl.pallas_call(
        _mm_body,
        grid=grid,
        in_specs=[
            pl.BlockSpec((bm, k), lambda i: (i, 0)),
            pl.BlockSpec((k, n), lambda i: (0, 0)),
        ],
        out_specs=pl.BlockSpec((bm, n), lambda i: (i, 0)),
        out_shape=jax.ShapeDtypeStruct((m, n), jnp.float32),
    )(x, w)


def kernel(x, edge_index, edge_weight, W1, b1, W2, b2):
    edge_weight = edge_weight.astype(jnp.float32)
    row, col = edge_index[0], edge_index[1]
    n = x.shape[0]

    deg = jax.ops.segment_sum(edge_weight, col, num_segments=n) + 1.0
    dis = jnp.where(deg > 0, jax.lax.rsqrt(deg), 0.0)

    xl = _matmul(dis[:, None] * x, W1)
    s1 = jax.ops.segment_sum(edge_weight[:, None] * jnp.take(xl, row, axis=0),
                             col, num_segments=n)
    h = jax.nn.elu(dis[:, None] * (s1 + xl) + b1)

    hl = _matmul(dis[:, None] * h, W2)
    s2 = jax.ops.segment_sum(edge_weight[:, None] * jnp.take(hl, row, axis=0),
                             col, num_segments=n)
    out = dis[:, None] * (s2 + hl) + b2
    return jax.nn.softplus(out) + 0.0001


# speedup vs baseline: 10.1554x; 3.8981x over previous
import functools
import jax
import jax.numpy as jnp
from jax import lax
from jax.experimental import pallas as pl
from jax.experimental.pallas import tpu as pltpu
from jax.experimental.pallas import tpu_sc as plsc

_NT = 16


def _splat_lane(vec16, i):
    idx = jnp.full((16, 1), i, jnp.int32)
    return lax.gather(
        vec16, idx,
        lax.GatherDimensionNumbers(offset_dims=(), collapsed_slice_dims=(0,),
                                   start_index_map=(0,)),
        (1,), mode=lax.GatherScatterMode.PROMISE_IN_BOUNDS)


def _make_deg(n, e, chunk):
    epc = e // _NT
    nchunks = epc // chunk
    rpt = n // _NT
    mesh = plsc.VectorSubcoreMesh(core_axis_name="c", subcore_axis_name="s")

    @functools.partial(
        pl.kernel,
        mesh=mesh,
        out_type=jax.ShapeDtypeStruct((n,), jnp.float32),
        scratch_types=[
            pltpu.VMEM_SHARED((n,), jnp.float32),
            pltpu.VMEM((rpt,), jnp.float32),
            pltpu.VMEM((chunk,), jnp.int32),
            pltpu.VMEM((chunk,), jnp.float32),
        ],
    )
    def deg_kernel(col_hbm, ew_hbm, zeros_hbm, out_hbm, acc, buf_v, col_v,
                   ew_v):
        c = lax.axis_index("c")
        s = lax.axis_index("s")

        @pl.when(c == 0)
        def _():
            b0 = pl.multiple_of(s * rpt, 8)
            pltpu.sync_copy(zeros_hbm.at[pl.ds(b0, rpt)], buf_v)
            pltpu.sync_copy(buf_v, acc.at[pl.ds(b0, rpt)])
            plsc.subcore_barrier()

            def do_chunk(k, carry):
                base = pl.multiple_of(s * epc + k * chunk, 8)
                pltpu.sync_copy(col_hbm.at[pl.ds(base, chunk)], col_v)
                pltpu.sync_copy(ew_hbm.at[pl.ds(base, chunk)], ew_v)
                pltpu.sync_copy(ew_v, acc.at[col_v], add=True)
                return carry

            lax.fori_loop(0, nchunks, do_chunk, 0)
            plsc.subcore_barrier()
            pltpu.sync_copy(acc.at[pl.ds(b0, rpt)], buf_v)
            pltpu.sync_copy(buf_v, out_hbm.at[pl.ds(b0, rpt)])

    return deg_kernel


def _make_spmm(n, e, d2, chunk, col_split):
    nw = _NT if col_split else 2 * _NT
    epc = e // nw
    nchunks = epc // chunk
    rpt = n // _NT
    io_chunk = rpt
    nio = 1
    while io_chunk > chunk:
        io_chunk //= 2
        nio *= 2
    assert nio * io_chunk == rpt and nchunks * chunk == epc
    out_shape = (n, 2 * d2) if col_split else (2, n, d2)
    mesh = plsc.VectorSubcoreMesh(core_axis_name="c", subcore_axis_name="s")

    @functools.partial(
        pl.kernel,
        mesh=mesh,
        out_type=jax.ShapeDtypeStruct(out_shape, jnp.float32),
        scratch_types=[
            pltpu.VMEM_SHARED((n, d2), jnp.float32),
            pltpu.VMEM((chunk, d2), jnp.float32),
            pltpu.VMEM((chunk,), jnp.int32),
            pltpu.VMEM((chunk,), jnp.int32),
            pltpu.VMEM((chunk,), jnp.int32),
            pltpu.VMEM((chunk,), jnp.float32),
            pltpu.SemaphoreType.DMA,
        ],
    )
    def spmm_kernel(table_hbm, row_hbm, col_hbm, ew_hbm, zeros_hbm, out_hbm,
                    acc, rows_v, row_v, gidx_v, col_v, ew_v, sem):
        c = lax.axis_index("c")
        s = lax.axis_index("s")
        base_r = s * rpt

        def init(t, carry):
            b = pl.multiple_of(base_r + t * io_chunk, 8)
            pltpu.sync_copy(zeros_hbm.at[pl.ds(b, io_chunk)],
                            rows_v.at[pl.ds(0, io_chunk)])
            pltpu.sync_copy(rows_v.at[pl.ds(0, io_chunk)],
                            acc.at[pl.ds(b, io_chunk)])
            return carry

        lax.fori_loop(0, nio, init, 0)
        plsc.subcore_barrier()

        if col_split:
            edge0 = s * epc
        else:
            edge0 = (c * _NT + s) * epc

        def do_chunk(k, carry):
            base = pl.multiple_of(edge0 + k * chunk, 8)
            pltpu.sync_copy(row_hbm.at[pl.ds(base, chunk)], row_v)
            pltpu.sync_copy(col_hbm.at[pl.ds(base, chunk)], col_v)
            pltpu.sync_copy(ew_hbm.at[pl.ds(base, chunk)], ew_v)

            if col_split:
                def mkidx(g, carry2):
                    r16 = row_v[pl.ds(g * 16, 16)]
                    gidx_v[pl.ds(g * 16, 16)] = r16 * 2 + c
                    return carry2

                lax.fori_loop(0, chunk // 16, mkidx, 0)
                gref = gidx_v
            else:
                gref = row_v
            pltpu.async_copy(table_hbm.at[gref], rows_v, sem).wait()

            def scale(g, carry2):
                w16 = ew_v[pl.ds(g * 16, 16)]
                for i in range(16):
                    eidx = g * 16 + i
                    wi = _splat_lane(w16, i)
                    for j in range(d2 // 16):
                        sl = pl.ds(j * 16, 16)
                        rows_v[eidx, sl] = rows_v[eidx, sl] * wi
                return carry2

            lax.fori_loop(0, chunk // 16, scale, 0)
            pltpu.sync_copy(rows_v, acc.at[col_v], add=True)
            return carry

        lax.fori_loop(0, nchunks, do_chunk, 0)
        plsc.subcore_barrier()

        def drain(t, carry):
            b = pl.multiple_of(base_r + t * io_chunk, 8)
            pltpu.sync_copy(acc.at[pl.ds(b, io_chunk)],
                            rows_v.at[pl.ds(0, io_chunk)])
            if col_split:
                lane0 = pl.multiple_of(c * d2, d2)
                dst = out_hbm.at[pl.ds(b, io_chunk), pl.ds(lane0, d2)]
            else:
                dst = out_hbm.at[c, pl.ds(b, io_chunk)]
            pltpu.sync_copy(rows_v.at[pl.ds(0, io_chunk)], dst)
            return carry

        lax.fori_loop(0, nio, drain, 0)

    return spmm_kernel


def _mm_pre_body(deg_ref, x_ref, w_ref, o_ref):
    dis = lax.rsqrt(deg_ref[...] + 1.0)
    o_ref[...] = jnp.dot(x_ref[...] * dis, w_ref[...],
                         preferred_element_type=jnp.float32)


def _mm_mid_body(deg_ref, s_ref, xl_ref, b_ref, w_ref, o_ref):
    dis = lax.rsqrt(deg_ref[...] + 1.0)
    pre = dis * (s_ref[...] + xl_ref[...]) + b_ref[...]
    h = jnp.where(pre > 0, pre, jnp.exp(jnp.minimum(pre, 0.0)) - 1.0)
    o_ref[...] = jnp.dot(h * dis, w_ref[...],
                         preferred_element_type=jnp.float32)


def _post_body(deg_ref, sa_ref, sb_ref, hl_ref, b_ref, o_ref):
    dis = lax.rsqrt(deg_ref[...] + 1.0)
    z = dis * (sa_ref[0] + sb_ref[0] + hl_ref[...]) + b_ref[...]
    o_ref[...] = (jnp.maximum(z, 0.0) + jnp.log(1.0 + jnp.exp(-jnp.abs(z)))
                  + 0.0001)


def _mm_pre(deg, x, w, bm):
    m, k = x.shape
    _, nn = w.shape
    return pl.pallas_call(
        _mm_pre_body,
        grid=(m // bm,),
        in_specs=[
            pl.BlockSpec((bm, 1), lambda i: (i, 0)),
            pl.BlockSpec((bm, k), lambda i: (i, 0)),
            pl.BlockSpec((k, nn), lambda i: (0, 0)),
        ],
        out_specs=pl.BlockSpec((bm, nn), lambda i: (i, 0)),
        out_shape=jax.ShapeDtypeStruct((m, nn), jnp.float32),
    )(deg, x, w)


def _mm_mid(deg, s, xl, b, w, bm):
    m, k = s.shape
    _, nn = w.shape
    return pl.pallas_call(
        _mm_mid_body,
        grid=(m // bm,),
        in_specs=[
            pl.BlockSpec((bm, 1), lambda i: (i, 0)),
            pl.BlockSpec((bm, k), lambda i: (i, 0)),
            pl.BlockSpec((bm, k), lambda i: (i, 0)),
            pl.BlockSpec((1, k), lambda i: (0, 0)),
            pl.BlockSpec((k, nn), lambda i: (0, 0)),
        ],
        out_specs=pl.BlockSpec((bm, nn), lambda i: (i, 0)),
        out_shape=jax.ShapeDtypeStruct((m, nn), jnp.float32),
    )(deg, s, xl, b, w)


def _post(deg, s2, hl, b, bm):
    m, k = hl.shape
    return pl.pallas_call(
        _post_body,
        grid=(m // bm,),
        in_specs=[
            pl.BlockSpec((bm, 1), lambda i: (i, 0)),
            pl.BlockSpec((1, bm, k), lambda i: (0, i, 0)),
            pl.BlockSpec((1, bm, k), lambda i: (1, i, 0)),
            pl.BlockSpec((bm, k), lambda i: (i, 0)),
            pl.BlockSpec((1, k), lambda i: (0, 0)),
        ],
        out_specs=pl.BlockSpec((bm, k), lambda i: (i, 0)),
        out_shape=jax.ShapeDtypeStruct((m, k), jnp.float32),
    )(deg, s2, s2, hl, b)


def kernel(x, edge_index, edge_weight, W1, b1, W2, b2):
    ew = edge_weight.astype(jnp.float32)
    row = edge_index[0].astype(jnp.int32)
    col = edge_index[1].astype(jnp.int32)
    n = x.shape[0]
    e = row.shape[0]
    d1 = W1.shape[1]
    d2o = W2.shape[1]

    npad = ((n + 255) // 256) * 256
    bm = npad // 8
    xp = jnp.pad(x, ((0, npad - n), (0, 0)))

    degz = jnp.zeros((npad,), jnp.float32)
    zeros1 = jnp.zeros((npad, d1 // 2), jnp.float32)
    zeros2 = jnp.zeros((npad, d2o), jnp.float32)

    deg = _make_deg(npad, e, 2000)(col, ew, degz)
    degc = deg.reshape(npad, 1)

    xl = _mm_pre(degc, xp, W1, bm)
    s1 = _make_spmm(npad, e, d1 // 2, 160, True)(
        xl.reshape(2 * npad, d1 // 2), row, col, ew, zeros1)
    hl = _mm_mid(degc, s1, xl, b1.reshape(1, d1), W2, bm)
    s2 = _make_spmm(npad, e, d2o, 80, False)(
        hl, row, col, ew, zeros2)
    return _post(degc, s2, hl, b2.reshape(1, d2o), bm)[:n]
